# baseline (speedup 1.0000x reference)
import functools

import jax
import jax.numpy as jnp
from jax import lax
from jax.experimental import pallas as pl
from jax.experimental.pallas import tpu as pltpu
from jax.experimental.pallas import tpu_sc as plsc

D = 64
W = 2 * D
NC, NS = 2, 16
NW = NC * NS
CH = 128


def _sc_mesh():
    return plsc.VectorSubcoreMesh(core_axis_name="c", subcore_axis_name="s",
                                  num_cores=NC, num_subcores=NS)


def _sc_gather_rows(uf2, mf2, uid_phys, mid_phys, batch):
    bpw = batch // NW
    nch = bpw // CH
    f32 = jnp.float32

    @functools.partial(
        pl.kernel,
        out_type=(
            jax.ShapeDtypeStruct((batch, W), f32),
            jax.ShapeDtypeStruct((batch, W), f32),
        ),
        mesh=_sc_mesh(),
        scratch_types=[
            pltpu.VMEM((nch, CH), jnp.int32),
            pltpu.VMEM((nch, CH), jnp.int32),
            pltpu.VMEM((bpw // 2, W), f32),
            pltpu.VMEM((bpw // 2, W), f32),
            pltpu.SemaphoreType.DMA,
        ],
    )
    def body(uf_hbm, mf_hbm, uid_hbm, mid_hbm, uo_hbm, mo_hbm,
             uidx, midx, urows, mrows, sem):
        wid = lax.axis_index("s") * NC + lax.axis_index("c")
        base = wid * bpw
        half = nch // 2
        for j in range(nch):
            pltpu.sync_copy(uid_hbm.at[pl.ds(base + j * CH, CH)], uidx.at[j])
            pltpu.sync_copy(mid_hbm.at[pl.ds(base + j * CH, CH)], midx.at[j])
        for h in range(2):
            copies = []
            for k in range(half):
                j = h * half + k
                sl = pl.ds(k * CH, CH)
                copies.append(pltpu.async_copy(uf_hbm.at[uidx.at[j]], urows.at[sl], sem))
                copies.append(pltpu.async_copy(mf_hbm.at[midx.at[j]], mrows.at[sl], sem))
            for c in copies:
                c.wait()
            out_sl = pl.ds(base + h * (bpw // 2), bpw // 2)
            pltpu.sync_copy(urows, uo_hbm.at[out_sl])
            pltpu.sync_copy(mrows, mo_hbm.at[out_sl])

    return body(uf2, mf2, uid_phys, mid_phys)


def _sc_gather_bias(ub1d, mb1d, uid, mid, batch):
    bpw = batch // NW
    nch = bpw // CH
    f32 = jnp.float32

    @functools.partial(
        pl.kernel,
        out_type=(
            jax.ShapeDtypeStruct((batch,), f32),
            jax.ShapeDtypeStruct((batch,), f32),
        ),
        mesh=_sc_mesh(),
        compiler_params=pltpu.CompilerParams(use_tc_tiling_on_sc=False),
        scratch_types=[
            pltpu.VMEM((nch, CH), jnp.int32),
            pltpu.VMEM((nch, CH), jnp.int32),
            pltpu.VMEM((bpw,), f32),
            pltpu.VMEM((bpw,), f32),
            pltpu.SemaphoreType.DMA,
        ],
    )
    def body(ub_hbm, mb_hbm, uid_hbm, mid_hbm, ubo_hbm, mbo_hbm,
             uidx, midx, ubv, mbv, sem):
        wid = lax.axis_index("s") * NC + lax.axis_index("c")
        base = wid * bpw
        for j in range(nch):
            pltpu.sync_copy(uid_hbm.at[pl.ds(base + j * CH, CH)], uidx.at[j])
            pltpu.sync_copy(mid_hbm.at[pl.ds(base + j * CH, CH)], midx.at[j])
        copies = []
        for j in range(nch):
            sl = pl.ds(j * CH, CH)
            copies.append(pltpu.async_copy(ub_hbm.at[uidx.at[j]], ubv.at[sl], sem))
            copies.append(pltpu.async_copy(mb_hbm.at[midx.at[j]], mbv.at[sl], sem))
        for c in copies:
            c.wait()
        out_sl = pl.ds(base, bpw)
        pltpu.sync_copy(ubv, ubo_hbm.at[out_sl])
        pltpu.sync_copy(mbv, mbo_hbm.at[out_sl])

    return body(ub1d, mb1d, uid, mid)


def _ln(x, w, b, eps=1e-5):
    mean = jnp.mean(x, axis=-1, keepdims=True)
    xc = x - mean
    var = jnp.mean(xc * xc, axis=-1, keepdims=True)
    return xc / jnp.sqrt(var + eps) * w + b


def _tc_body(uw_ref, mw_ref, ub_ref, mb_ref, uid_ref, mid_ref, aid_ref,
             af_ref, unw, unb, mnw, mnb, anw, anb, gb_ref, out_ref):
    blk = uw_ref.shape[0]
    u_odd = (uid_ref[...] & 1) == 1
    m_odd = (mid_ref[...] & 1) == 1
    u_raw = jnp.where(u_odd, uw_ref[:, D:], uw_ref[:, :D])
    m_raw = jnp.where(m_odd, mw_ref[:, D:], mw_ref[:, :D])
    u = _ln(jnp.maximum(u_raw, 0.0), unw[...], unb[...])
    m = _ln(jnp.maximum(m_raw, 0.0), mnw[...], mnb[...])
    a_tab = _ln(jnp.maximum(af_ref[...], 0.0), anw[...], anb[...])
    n_age = af_ref.shape[0]
    onehot = (aid_ref[...] == lax.broadcasted_iota(jnp.int32, (blk, n_age), 1)
              ).astype(jnp.float32)
    ages = jnp.dot(onehot, a_tab, preferred_element_type=jnp.float32)
    dot = jnp.sum(u * m * ages, axis=1, keepdims=True)
    preds = dot * 0.125 + ub_ref[...] + mb_ref[...] + gb_ref[...]
    out_ref[...] = jnp.clip(preds, -0.1, 1.1)


def _tc_compute(u_wide, m_wide, ub, mb, uid2d, mid2d, aid2d, age_factors,
                unw, unb, mnw, mnb, anw, anb, gb, batch, grid):
    blk = batch // grid
    n_age = age_factors.shape[0]
    wide_spec = pl.BlockSpec((blk, W), lambda i: (i, 0))
    col_spec = pl.BlockSpec((blk, 1), lambda i: (i, 0))
    par_spec = pl.BlockSpec((1, D), lambda i: (0, 0))
    return pl.pallas_call(
        _tc_body,
        grid=(grid,),
        in_specs=[
            wide_spec, wide_spec, col_spec, col_spec,
            col_spec, col_spec, col_spec,
            pl.BlockSpec((n_age, D), lambda i: (0, 0)),
            par_spec, par_spec, par_spec, par_spec, par_spec, par_spec,
            pl.BlockSpec((1, 1), lambda i: (0, 0)),
        ],
        out_specs=col_spec,
        out_shape=jax.ShapeDtypeStruct((batch, 1), jnp.float32),
    )(u_wide, m_wide, ub, mb, uid2d, mid2d, aid2d, age_factors,
      unw, unb, mnw, mnb, anw, anb, gb)


def kernel(user_ids, movie_ids, age_bucket_ids,
           user_factors, movie_factors, age_factors,
           user_norm_w, user_norm_b, movie_norm_w, movie_norm_b,
           age_norm_w, age_norm_b, user_bias, movie_bias, global_bias):
    batch = user_ids.shape[0]
    uid = user_ids.astype(jnp.int32)
    mid = movie_ids.astype(jnp.int32)
    uf2 = user_factors.reshape(user_factors.shape[0] // 2, W)
    mf2 = movie_factors.reshape(movie_factors.shape[0] // 2, W)
    u_wide, m_wide = _sc_gather_rows(uf2, mf2, uid >> 1, mid >> 1, batch)
    ub, mb = _sc_gather_bias(user_bias.reshape(-1), movie_bias.reshape(-1),
                             uid, mid, batch)
    preds = _tc_compute(
        u_wide, m_wide, ub.reshape(batch, 1), mb.reshape(batch, 1),
        uid.reshape(batch, 1), mid.reshape(batch, 1),
        age_bucket_ids.astype(jnp.int32).reshape(batch, 1), age_factors,
        user_norm_w.reshape(1, D), user_norm_b.reshape(1, D),
        movie_norm_w.reshape(1, D), movie_norm_b.reshape(1, D),
        age_norm_w.reshape(1, D), age_norm_b.reshape(1, D),
        global_bias.reshape(1, 1), batch, grid=8)
    return preds.reshape(batch)


# speedup vs baseline: 1.1370x; 1.1370x over previous
import functools

import jax
import jax.numpy as jnp
from jax import lax
from jax.experimental import pallas as pl
from jax.experimental.pallas import tpu as pltpu
from jax.experimental.pallas import tpu_sc as plsc

D = 64
W = 2 * D
NC, NS = 2, 16
NW = NC * NS
CH = 128


def _sc_mesh():
    return plsc.VectorSubcoreMesh(core_axis_name="c", subcore_axis_name="s",
                                  num_cores=NC, num_subcores=NS)


def _sc_gather_rows(uf2, mf2, uid_phys, mid_phys, batch):
    bpw = batch // NW
    nch = bpw // CH
    f32 = jnp.float32

    @functools.partial(
        pl.kernel,
        out_type=(
            jax.ShapeDtypeStruct((batch, W), f32),
            jax.ShapeDtypeStruct((batch, W), f32),
        ),
        mesh=_sc_mesh(),
        scratch_types=[
            pltpu.VMEM((nch, CH), jnp.int32),
            pltpu.VMEM((nch, CH), jnp.int32),
            pltpu.VMEM((bpw // 2, W), f32),
            pltpu.VMEM((bpw // 2, W), f32),
            pltpu.SemaphoreType.DMA,
        ],
    )
    def body(uf_hbm, mf_hbm, uid_hbm, mid_hbm, uo_hbm, mo_hbm,
             uidx, midx, urows, mrows, sem):
        wid = lax.axis_index("s") * NC + lax.axis_index("c")
        base = wid * bpw
        half = nch // 2
        for j in range(nch):
            pltpu.sync_copy(uid_hbm.at[pl.ds(base + j * CH, CH)], uidx.at[j])
            pltpu.sync_copy(mid_hbm.at[pl.ds(base + j * CH, CH)], midx.at[j])
        for h in range(2):
            copies = []
            for k in range(half):
                j = h * half + k
                sl = pl.ds(k * CH, CH)
                copies.append(pltpu.async_copy(uf_hbm.at[uidx.at[j]], urows.at[sl], sem))
                copies.append(pltpu.async_copy(mf_hbm.at[midx.at[j]], mrows.at[sl], sem))
            for c in copies:
                c.wait()
            out_sl = pl.ds(base + h * (bpw // 2), bpw // 2)
            pltpu.sync_copy(urows, uo_hbm.at[out_sl])
            pltpu.sync_copy(mrows, mo_hbm.at[out_sl])

    return body(uf2, mf2, uid_phys, mid_phys)


def _sc_gather_bias(ub1d, mb1d, uid, mid, batch):
    bpw = batch // NW
    nch = bpw // CH
    f32 = jnp.float32

    @functools.partial(
        pl.kernel,
        out_type=(
            jax.ShapeDtypeStruct((batch,), f32),
            jax.ShapeDtypeStruct((batch,), f32),
        ),
        mesh=_sc_mesh(),
        compiler_params=pltpu.CompilerParams(use_tc_tiling_on_sc=False),
        scratch_types=[
            pltpu.VMEM((nch, CH), jnp.int32),
            pltpu.VMEM((nch, CH), jnp.int32),
            pltpu.VMEM((bpw,), f32),
            pltpu.VMEM((bpw,), f32),
            pltpu.SemaphoreType.DMA,
        ],
    )
    def body(ub_hbm, mb_hbm, uid_hbm, mid_hbm, ubo_hbm, mbo_hbm,
             uidx, midx, ubv, mbv, sem):
        wid = lax.axis_index("s") * NC + lax.axis_index("c")
        base = wid * bpw
        for j in range(nch):
            pltpu.sync_copy(uid_hbm.at[pl.ds(base + j * CH, CH)], uidx.at[j])
            pltpu.sync_copy(mid_hbm.at[pl.ds(base + j * CH, CH)], midx.at[j])
        copies = []
        for j in range(nch):
            sl = pl.ds(j * CH, CH)
            copies.append(pltpu.async_copy(ub_hbm.at[uidx.at[j]], ubv.at[sl], sem))
            copies.append(pltpu.async_copy(mb_hbm.at[midx.at[j]], mbv.at[sl], sem))
        for c in copies:
            c.wait()
        out_sl = pl.ds(base, bpw)
        pltpu.sync_copy(ubv, ubo_hbm.at[out_sl])
        pltpu.sync_copy(mbv, mbo_hbm.at[out_sl])

    return body(ub1d, mb1d, uid, mid)


def _ln(x, w, b, eps=1e-5):
    mean = jnp.mean(x, axis=-1, keepdims=True)
    xc = x - mean
    var = jnp.mean(xc * xc, axis=-1, keepdims=True)
    return xc / jnp.sqrt(var + eps) * w + b


def _tc_body(uw_ref, mw_ref, ub_ref, mb_ref, aid_ref,
             af_ref, unw, unb, mnw, mnb, anw, anb, gb_ref, out_ref):
    blk = uw_ref.shape[0]
    u_raw = uw_ref[:, :D]
    m_raw = mw_ref[:, :D]
    u = _ln(jnp.maximum(u_raw, 0.0), unw[...], unb[...])
    m = _ln(jnp.maximum(m_raw, 0.0), mnw[...], mnb[...])
    a_tab = _ln(jnp.maximum(af_ref[...], 0.0), anw[...], anb[...])
    n_age = af_ref.shape[0]
    onehot = (aid_ref[...] == lax.broadcasted_iota(jnp.int32, (blk, n_age), 1)
              ).astype(jnp.float32)
    ages = jnp.dot(onehot, a_tab, preferred_element_type=jnp.float32)
    dot = jnp.sum(u * m * ages, axis=1, keepdims=True)
    preds = dot * 0.125 + ub_ref[...] + mb_ref[...] + gb_ref[...]
    out_ref[...] = jnp.clip(preds, -0.1, 1.1)


def _tc_compute(u_wide, m_wide, ub, mb, aid2d, age_factors,
                unw, unb, mnw, mnb, anw, anb, gb, batch, grid):
    blk = batch // grid
    n_age = age_factors.shape[0]
    wide_spec = pl.BlockSpec((blk, W), lambda i: (i, 0))
    col_spec = pl.BlockSpec((blk, 1), lambda i: (i, 0))
    par_spec = pl.BlockSpec((1, D), lambda i: (0, 0))
    return pl.pallas_call(
        _tc_body,
        grid=(grid,),
        in_specs=[
            wide_spec, wide_spec, col_spec, col_spec, col_spec,
            pl.BlockSpec((n_age, D), lambda i: (0, 0)),
            par_spec, par_spec, par_spec, par_spec, par_spec, par_spec,
            pl.BlockSpec((1, 1), lambda i: (0, 0)),
        ],
        out_specs=col_spec,
        out_shape=jax.ShapeDtypeStruct((batch, 1), jnp.float32),
    )(u_wide, m_wide, ub, mb, aid2d, age_factors,
      unw, unb, mnw, mnb, anw, anb, gb)


def kernel(user_ids, movie_ids, age_bucket_ids,
           user_factors, movie_factors, age_factors,
           user_norm_w, user_norm_b, movie_norm_w, movie_norm_b,
           age_norm_w, age_norm_b, user_bias, movie_bias, global_bias):
    batch = user_ids.shape[0]
    uid = user_ids.astype(jnp.int32)
    mid = movie_ids.astype(jnp.int32)
    uf2 = jnp.pad(user_factors, ((0, 0), (0, W - D)))
    mf2 = jnp.pad(movie_factors, ((0, 0), (0, W - D)))
    u_wide, m_wide = _sc_gather_rows(uf2, mf2, uid, mid, batch)
    ub, mb = _sc_gather_bias(user_bias.reshape(-1), movie_bias.reshape(-1),
                             uid, mid, batch)
    preds = _tc_compute(
        u_wide, m_wide, ub.reshape(batch, 1), mb.reshape(batch, 1),
        age_bucket_ids.astype(jnp.int32).reshape(batch, 1), age_factors,
        user_norm_w.reshape(1, D), user_norm_b.reshape(1, D),
        movie_norm_w.reshape(1, D), movie_norm_b.reshape(1, D),
        age_norm_w.reshape(1, D), age_norm_b.reshape(1, D),
        global_bias.reshape(1, 1), batch, grid=8)
    return preds.reshape(batch)
